# Initial kernel scaffold; baseline (speedup 1.0000x reference)
import jax
import jax.numpy as jnp
from jax.experimental import pallas as pl


def kernel(feats, level_start_idx, spatial_shapes, embed_weight):
    raise NotImplementedError("write your pallas kernel here")


# speedup vs baseline: 5.9276x; 5.9276x over previous
import jax
import jax.numpy as jnp
from jax.experimental import pallas as pl
from jax.experimental.pallas import tpu as pltpu


def _tc_body(feats_ref, embed_ref, out_ref):
    B = feats_ref.shape[1]
    C = feats_ref.shape[2]
    hb = feats_ref.shape[3]
    W = feats_ref.shape[4]
    e = embed_ref[0]
    for b in range(B):
        x = feats_ref[0, b].reshape(C, hb * W)
        out_ref[:, b, :] = x.T + e


def kernel(feats, level_start_idx, spatial_shapes, embed_weight):
    L, B, C, H, W = feats.shape
    hb = 8
    n_hblk = H // hb
    grid = (L, n_hblk)

    out = pl.pallas_call(
        _tc_body,
        grid=grid,
        in_specs=[
            pl.BlockSpec((1, B, C, hb, W), lambda l, j: (l, 0, 0, j, 0)),
            pl.BlockSpec((1, 1, C), lambda l, j: (l, 0, 0)),
        ],
        out_specs=pl.BlockSpec(
            (hb * W, B, C), lambda l, j: (l * (H // hb) + j, 0, 0)
        ),
        out_shape=jax.ShapeDtypeStruct((L * H * W, B, C), feats.dtype),
    )(feats, embed_weight.reshape(L, 1, C))
    return out
